# baseline (speedup 1.0000x reference)
import functools

import jax
import jax.numpy as jnp
from jax import lax
from jax.experimental import pallas as pl
from jax.experimental.pallas import tpu as pltpu
from jax.experimental.pallas import tpu_sc as plsc

B = 16384
N = 200
D = 64
R = 6.0

_info = plsc.get_sparse_core_info()
NC, NS, L = _info.num_cores, _info.num_subcores, _info.num_lanes
NW = NC * NS
BPW = B // NW
TPG = 128
NTG = BPW // TPG
IC = 64
NCH = BPW // IC
NBUF = 4
H1 = 128
H2 = N - H1

_C = (1.0000006974281586, -0.5000073548516979, 0.3331793391436614,
      -0.2492950419943796, 0.2045542018978282, -0.1845583495672427,
      0.11784427706676123)
_SQRT2 = 1.4142135623730951
_LN2 = 0.6931471805599453

_mesh = plsc.VectorSubcoreMesh(core_axis_name="c", subcore_axis_name="s")

_FBM = 2048


_WV = 1000000
_SBM = 20000


def _scale_body(t_ref, o_ref):
    o_ref[...] = t_ref[...] * R


def _scale_tc(word_table):
    return pl.pallas_call(
        _scale_body,
        grid=(_WV // _SBM,),
        in_specs=[pl.BlockSpec((_SBM, D), lambda i: (i, 0))],
        out_specs=pl.BlockSpec((_SBM, D), lambda i: (i, 0)),
        out_shape=jax.ShapeDtypeStruct((_WV, D), jnp.float32),
    )(word_table)


def _split_body(w_ref, a_ref, b_ref):
    a_ref[...] = w_ref[:, :H1]
    b_ref[...] = jnp.zeros((_FBM, 128), jnp.int32)
    b_ref[:, :H2] = w_ref[:, H1:]


def _split_tc(words):
    return pl.pallas_call(
        _split_body,
        grid=(B // _FBM,),
        in_specs=[pl.BlockSpec((_FBM, N), lambda i: (i, 0))],
        out_specs=[pl.BlockSpec((_FBM, 128), lambda i: (i, 0)),
                   pl.BlockSpec((_FBM, 128), lambda i: (i, 0))],
        out_shape=[jax.ShapeDtypeStruct((B, 128), jnp.int32),
                   jax.ShapeDtypeStruct((B, 128), jnp.int32)],
    )(words)


def _log_over_r(a):
    bits = plsc.bitcast(a, jnp.int32)
    e = lax.shift_right_logical(bits, 23) - 127
    m = plsc.bitcast((bits & 0x007FFFFF) | 0x3F800000, jnp.float32)
    adj = m >= _SQRT2
    m = jnp.where(adj, m * 0.5, m)
    e = (e + adj.astype(jnp.int32)).astype(jnp.float32)
    t = m - 1.0
    p = jnp.float32(_C[6])
    for k in range(5, -1, -1):
        p = p * t + _C[k]
    return (e * _LN2 + t * p) * (1.0 / R)


@functools.partial(
    pl.kernel,
    out_type=jax.ShapeDtypeStruct((B, 2 * D), jnp.float32),
    mesh=_mesh,
    compiler_params=pltpu.CompilerParams(
        use_tc_tiling_on_sc=False, needs_layout_passes=False),
    scratch_types=[
        pltpu.VMEM((IC, 128), jnp.int32),
        pltpu.VMEM((IC, 128), jnp.int32),
        pltpu.VMEM((NBUF, N, D), jnp.float32),
        pltpu.VMEM((IC, D), jnp.float32),
        pltpu.VMEM((TPG,), jnp.int32),
        pltpu.VMEM((TPG, D), jnp.float32),
        pltpu.SemaphoreType.DMA,
        pltpu.SemaphoreType.DMA,
        pltpu.SemaphoreType.DMA,
        pltpu.SemaphoreType.DMA,
        pltpu.SemaphoreType.DMA,
    ],
)
def _sc_emb(wa_hbm, wb_hbm, tag_hbm, wtab_hbm, ttab_hbm, out_hbm,
            icha, ichb, rows_v, out_v, tidx_v, trows_v, s0, s1, s2, s3, tsem):
    sems = (s0, s1, s2, s3)
    wid = lax.axis_index("s") * NC + lax.axis_index("c")
    base = wid * BPW

    for t in range(NTG):
        pltpu.sync_copy(tag_hbm.at[pl.ds(base + t * TPG, TPG)], tidx_v)
        pltpu.async_copy(ttab_hbm.at[tidx_v], trows_v, tsem).wait()
        pltpu.sync_copy(
            trows_v, out_hbm.at[pl.ds(base + t * TPG, TPG), pl.ds(0, D)])

    def fire(s, local):
        pltpu.async_copy(wtab_hbm.at[icha.at[local]],
                         rows_v.at[s, pl.ds(0, H1)], sems[s])
        pltpu.async_copy(wtab_hbm.at[ichb.at[local, pl.ds(0, H2)]],
                         rows_v.at[s, pl.ds(H1, H2)], sems[s])

    @pl.loop(0, NCH)
    def _chunk(c):
        cb = base + c * IC
        pltpu.sync_copy(wa_hbm.at[pl.ds(cb, IC)], icha)
        pltpu.sync_copy(wb_hbm.at[pl.ds(cb, IC)], ichb)
        for s in range(NBUF):
            fire(s, s)

        @pl.loop(0, IC, step=NBUF)
        def _group(l):
            for s in range(NBUF):
                local = l + s
                pltpu.make_async_copy(
                    wtab_hbm.at[pl.ds(0, N)], rows_v.at[s], sems[s]).wait()

                zeros = jnp.zeros((L,), jnp.float32)

                @pl.loop(0, N, step=2, init_carry=(zeros, zeros, zeros, zeros))
                def _row(n, carry):
                    a0, a1, a2, a3 = carry
                    for dn in range(2):
                        a0 = a0 + jnp.exp(rows_v[s, n + dn, pl.ds(0 * L, L)])
                        a1 = a1 + jnp.exp(rows_v[s, n + dn, pl.ds(1 * L, L)])
                        a2 = a2 + jnp.exp(rows_v[s, n + dn, pl.ds(2 * L, L)])
                        a3 = a3 + jnp.exp(rows_v[s, n + dn, pl.ds(3 * L, L)])
                    return a0, a1, a2, a3

                a0, a1, a2, a3 = _row
                out_v[local, pl.ds(0 * L, L)] = _log_over_r(a0)
                out_v[local, pl.ds(1 * L, L)] = _log_over_r(a1)
                out_v[local, pl.ds(2 * L, L)] = _log_over_r(a2)
                out_v[local, pl.ds(3 * L, L)] = _log_over_r(a3)

                @pl.when(local + NBUF < IC)
                def _():
                    fire(s, local + NBUF)

        pltpu.sync_copy(out_v, out_hbm.at[pl.ds(cb, IC), pl.ds(D, D)])


@jax.jit
def kernel(words, tag, word_table, tag_table):
    wa, wb = _split_tc(words)
    return _sc_emb(wa, wb, tag, _scale_tc(word_table), tag_table)


# speedup vs baseline: 1.3801x; 1.3801x over previous
import functools

import jax
import jax.numpy as jnp
from jax import lax
from jax.experimental import pallas as pl
from jax.experimental.pallas import tpu as pltpu
from jax.experimental.pallas import tpu_sc as plsc

B = 16384
N = 200
D = 64
R = 6.0

_info = plsc.get_sparse_core_info()
NC, NS, L = _info.num_cores, _info.num_subcores, _info.num_lanes
NW = NC * NS
BPW = B // NW
TPG = 128
NTG = BPW // TPG
IC = 64
NCH = BPW // IC
NBUF = 4
H1 = 128
H2 = N - H1

_C = (1.0000006974281586, -0.5000073548516979, 0.3331793391436614,
      -0.2492950419943796, 0.2045542018978282, -0.1845583495672427,
      0.11784427706676123)
_SQRT2 = 1.4142135623730951
_LN2 = 0.6931471805599453

_mesh = plsc.VectorSubcoreMesh(core_axis_name="c", subcore_axis_name="s")

_FBM = 2048


_WV = 1000000
_SBM = 20000


def _scale_body(t_ref, o_ref):
    o_ref[...] = t_ref[...] * R


def _scale_tc(word_table):
    return pl.pallas_call(
        _scale_body,
        grid=(_WV // _SBM,),
        in_specs=[pl.BlockSpec((_SBM, D), lambda i: (i, 0))],
        out_specs=pl.BlockSpec((_SBM, D), lambda i: (i, 0)),
        out_shape=jax.ShapeDtypeStruct((_WV, D), jnp.float32),
    )(word_table)


def _split_body(w_ref, a_ref, b_ref):
    a_ref[...] = w_ref[:, :H1]
    b_ref[...] = jnp.zeros((_FBM, 128), jnp.int32)
    b_ref[:, :H2] = w_ref[:, H1:]


def _split_tc(words):
    return pl.pallas_call(
        _split_body,
        grid=(B // _FBM,),
        in_specs=[pl.BlockSpec((_FBM, N), lambda i: (i, 0))],
        out_specs=[pl.BlockSpec((_FBM, 128), lambda i: (i, 0)),
                   pl.BlockSpec((_FBM, 128), lambda i: (i, 0))],
        out_shape=[jax.ShapeDtypeStruct((B, 128), jnp.int32),
                   jax.ShapeDtypeStruct((B, 128), jnp.int32)],
    )(words)


def _log_over_r(a):
    bits = plsc.bitcast(a, jnp.int32)
    e = lax.shift_right_logical(bits, 23) - 127
    m = plsc.bitcast((bits & 0x007FFFFF) | 0x3F800000, jnp.float32)
    adj = m >= _SQRT2
    m = jnp.where(adj, m * 0.5, m)
    e = (e + adj.astype(jnp.int32)).astype(jnp.float32)
    t = m - 1.0
    p = jnp.float32(_C[6])
    for k in range(5, -1, -1):
        p = p * t + _C[k]
    return (e * _LN2 + t * p) * (1.0 / R)


@functools.partial(
    pl.kernel,
    out_type=jax.ShapeDtypeStruct((B, 2 * D), jnp.float32),
    mesh=_mesh,
    compiler_params=pltpu.CompilerParams(
        use_tc_tiling_on_sc=False, needs_layout_passes=False),
    scratch_types=[
        pltpu.VMEM((IC, 128), jnp.int32),
        pltpu.VMEM((IC, 128), jnp.int32),
        pltpu.VMEM((NBUF, N, D), jnp.float32),
        pltpu.VMEM((IC, D), jnp.float32),
        pltpu.VMEM((TPG,), jnp.int32),
        pltpu.VMEM((TPG, D), jnp.float32),
        pltpu.SemaphoreType.DMA,
        pltpu.SemaphoreType.DMA,
        pltpu.SemaphoreType.DMA,
        pltpu.SemaphoreType.DMA,
        pltpu.SemaphoreType.DMA,
    ],
)
def _sc_emb(wa_hbm, wb_hbm, tag_hbm, wtab_hbm, ttab_hbm, out_hbm,
            icha, ichb, rows_v, out_v, tidx_v, trows_v, s0, s1, s2, s3, tsem):
    sems = (s0, s1, s2, s3)
    wid = lax.axis_index("s") * NC + lax.axis_index("c")
    base = wid * BPW

    for t in range(NTG):
        pltpu.sync_copy(tag_hbm.at[pl.ds(base + t * TPG, TPG)], tidx_v)
        pltpu.async_copy(ttab_hbm.at[tidx_v], trows_v, tsem).wait()
        pltpu.sync_copy(
            trows_v, out_hbm.at[pl.ds(base + t * TPG, TPG), pl.ds(0, D)])

    def fire(s, local):
        pltpu.async_copy(wtab_hbm.at[icha.at[local]],
                         rows_v.at[s, pl.ds(0, H1)], sems[s])
        pltpu.async_copy(wtab_hbm.at[ichb.at[local, pl.ds(0, H2)]],
                         rows_v.at[s, pl.ds(H1, H2)], sems[s])

    @pl.loop(0, NCH)
    def _chunk(c):
        cb = base + c * IC
        pltpu.sync_copy(wa_hbm.at[pl.ds(cb, IC)], icha)
        pltpu.sync_copy(wb_hbm.at[pl.ds(cb, IC)], ichb)
        for s in range(NBUF):
            fire(s, s)

        @pl.loop(0, IC, step=NBUF)
        def _group(l):
            for s in range(NBUF):
                local = l + s
                pltpu.make_async_copy(
                    wtab_hbm.at[pl.ds(0, N)], rows_v.at[s], sems[s]).wait()

                zeros = jnp.zeros((L,), jnp.float32)

                @pl.loop(0, N, step=2, init_carry=(zeros, zeros, zeros, zeros))
                def _row(n, carry):
                    a0, a1, a2, a3 = carry
                    for dn in range(2):
                        a0 = a0 + jnp.exp(rows_v[s, n + dn, pl.ds(0 * L, L)] * R)
                        a1 = a1 + jnp.exp(rows_v[s, n + dn, pl.ds(1 * L, L)] * R)
                        a2 = a2 + jnp.exp(rows_v[s, n + dn, pl.ds(2 * L, L)] * R)
                        a3 = a3 + jnp.exp(rows_v[s, n + dn, pl.ds(3 * L, L)] * R)
                    return a0, a1, a2, a3

                a0, a1, a2, a3 = _row
                out_v[local, pl.ds(0 * L, L)] = _log_over_r(a0)
                out_v[local, pl.ds(1 * L, L)] = _log_over_r(a1)
                out_v[local, pl.ds(2 * L, L)] = _log_over_r(a2)
                out_v[local, pl.ds(3 * L, L)] = _log_over_r(a3)

                @pl.when(local + NBUF < IC)
                def _():
                    fire(s, local + NBUF)

        pltpu.sync_copy(out_v, out_hbm.at[pl.ds(cb, IC), pl.ds(D, D)])


@jax.jit
def kernel(words, tag, word_table, tag_table):
    wa, wb = _split_tc(words)
    return _sc_emb(wa, wb, tag, word_table, tag_table)
